# baseline (speedup 1.0000x reference)
import functools

import jax
import jax.numpy as jnp
from jax import lax
from jax.experimental import pallas as pl
from jax.experimental.pallas import tpu as pltpu
from jax.experimental.pallas import tpu_sc as plsc

_NUM_CORES = 2
_NUM_SUBCORES = 16
_NUM_WORKERS = _NUM_CORES * _NUM_SUBCORES
_CHUNK_ROWS = 32


def kernel(x, pos_table):
    batch, seq_len = x.shape
    d_model = pos_table.shape[1]
    pos = pos_table[:seq_len]
    rows_per_w = seq_len // _NUM_WORKERS
    n_chunks = rows_per_w // _CHUNK_ROWS
    mesh = plsc.VectorSubcoreMesh(
        core_axis_name="c", subcore_axis_name="s",
        num_cores=_NUM_CORES, num_subcores=_NUM_SUBCORES)

    @functools.partial(
        pl.kernel,
        out_type=jax.ShapeDtypeStruct((batch * seq_len, d_model), pos_table.dtype),
        mesh=mesh,
        scratch_types=[
            pltpu.SemaphoreType.DMA((2,)),
        ],
    )
    def copy_kernel(pos_hbm, out_hbm, sems):
        wid = lax.axis_index("s") * _NUM_CORES + lax.axis_index("c")
        base = wid * rows_per_w
        rows = pl.ds(base, rows_per_w)
        handles = [
            pltpu.async_copy(pos_hbm.at[rows],
                             out_hbm.at[pl.ds(b * seq_len + base, rows_per_w)],
                             sems.at[b])
            for b in range(batch)
        ]
        for h in handles:
            h.wait()

    flat = copy_kernel(pos)
    return flat.reshape(batch, seq_len, d_model)


# speedup vs baseline: 37.7284x; 37.7284x over previous
import functools

import jax
import jax.numpy as jnp
from jax import lax
from jax.experimental import pallas as pl
from jax.experimental.pallas import tpu as pltpu
from jax.experimental.pallas import tpu_sc as plsc

_NUM_CORES = 2
_NUM_SUBCORES = 16
_NUM_WORKERS = _NUM_CORES * _NUM_SUBCORES
_CHUNK_ROWS = 32
_NBUF = 3


def kernel(x, pos_table):
    batch, seq_len = x.shape
    d_model = pos_table.shape[1]
    pos = pos_table[:seq_len]
    rows_per_w = seq_len // _NUM_WORKERS
    n_chunks = rows_per_w // _CHUNK_ROWS
    mesh = plsc.VectorSubcoreMesh(
        core_axis_name="c", subcore_axis_name="s",
        num_cores=_NUM_CORES, num_subcores=_NUM_SUBCORES)

    @functools.partial(
        pl.kernel,
        out_type=jax.ShapeDtypeStruct((batch * seq_len, d_model), pos_table.dtype),
        mesh=mesh,
        scratch_types=[
            pltpu.VMEM((_NBUF, _CHUNK_ROWS, d_model), jnp.float32),
            pltpu.SemaphoreType.DMA((_NBUF,)),
            pltpu.SemaphoreType.DMA((_NBUF, 2)),
        ],
    )
    def copy_kernel(pos_hbm, out_hbm, bufs, in_sems, out_sems):
        wid = lax.axis_index("s") * _NUM_CORES + lax.axis_index("c")
        base = wid * rows_per_w

        def in_copy(c):
            slot = c % _NBUF
            return pltpu.make_async_copy(
                pos_hbm.at[pl.ds(base + c * _CHUNK_ROWS, _CHUNK_ROWS)],
                bufs.at[slot], in_sems.at[slot])

        def out_copy(c, b):
            slot = c % _NBUF
            return pltpu.make_async_copy(
                bufs.at[slot],
                out_hbm.at[pl.ds(b * seq_len + base + c * _CHUNK_ROWS, _CHUNK_ROWS)],
                out_sems.at[slot, b])

        for c in range(min(_NBUF, n_chunks)):
            in_copy(c).start()
        for c in range(n_chunks):
            in_copy(c).wait()
            for b in range(batch):
                out_copy(c, b).start()
            if c + _NBUF < n_chunks:
                for b in range(batch):
                    out_copy(c, b).wait()
                in_copy(c + _NBUF).start()
        for c in range(max(0, n_chunks - _NBUF), n_chunks):
            for b in range(batch):
                out_copy(c, b).wait()

    flat = copy_kernel(pos)
    return flat.reshape(batch, seq_len, d_model)


# speedup vs baseline: 65.5630x; 1.7378x over previous
import jax
import jax.numpy as jnp
from jax.experimental import pallas as pl
from jax.experimental.pallas import tpu as pltpu


_N_CHUNKS = 32


def _copy_body(pos_hbm, out_hbm, buf, in_sems, out_sems):
    batch = out_hbm.shape[0]
    seq_len = pos_hbm.shape[0]
    chunk = seq_len // _N_CHUNKS

    def in_copy(c):
        rows = pl.ds(c * chunk, chunk)
        return pltpu.make_async_copy(pos_hbm.at[rows], buf.at[rows], in_sems.at[c])

    def out_copy(c, b):
        rows = pl.ds(c * chunk, chunk)
        return pltpu.make_async_copy(buf.at[rows], out_hbm.at[b, rows], out_sems.at[c, b])

    for c in range(_N_CHUNKS):
        in_copy(c).start()
    for c in range(_N_CHUNKS):
        in_copy(c).wait()
        for b in range(batch):
            out_copy(c, b).start()
    for c in range(_N_CHUNKS):
        for b in range(batch):
            out_copy(c, b).wait()


def kernel(x, pos_table):
    batch, seq_len = x.shape
    d_model = pos_table.shape[1]
    pos = pos_table[:seq_len]
    return pl.pallas_call(
        _copy_body,
        in_specs=[pl.BlockSpec(memory_space=pl.ANY)],
        out_specs=pl.BlockSpec(memory_space=pl.ANY),
        out_shape=jax.ShapeDtypeStruct((batch, seq_len, d_model), pos_table.dtype),
        scratch_shapes=[
            pltpu.VMEM((seq_len, d_model), pos_table.dtype),
            pltpu.SemaphoreType.DMA((_N_CHUNKS,)),
            pltpu.SemaphoreType.DMA((_N_CHUNKS, batch)),
        ],
    )(pos)


# speedup vs baseline: 67.8544x; 1.0349x over previous
import jax
import jax.numpy as jnp
from jax.experimental import pallas as pl
from jax.experimental.pallas import tpu as pltpu


_N_CHUNKS = 4


def _copy_body(pos_hbm, out_hbm, buf, in_sems, out_sems):
    batch = out_hbm.shape[0]
    seq_len = pos_hbm.shape[0]
    chunk = seq_len // _N_CHUNKS

    def in_copy(c):
        rows = pl.ds(c * chunk, chunk)
        return pltpu.make_async_copy(pos_hbm.at[rows], buf.at[rows], in_sems.at[c])

    def out_copy(c, b):
        rows = pl.ds(c * chunk, chunk)
        return pltpu.make_async_copy(buf.at[rows], out_hbm.at[b, rows], out_sems.at[c, b])

    for c in range(_N_CHUNKS):
        in_copy(c).start()
    for c in range(_N_CHUNKS):
        in_copy(c).wait()
        for b in range(batch):
            out_copy(c, b).start()
    for c in range(_N_CHUNKS):
        for b in range(batch):
            out_copy(c, b).wait()


def kernel(x, pos_table):
    batch, seq_len = x.shape
    d_model = pos_table.shape[1]
    pos = pos_table[:seq_len]
    return pl.pallas_call(
        _copy_body,
        in_specs=[pl.BlockSpec(memory_space=pl.ANY)],
        out_specs=pl.BlockSpec(memory_space=pl.ANY),
        out_shape=jax.ShapeDtypeStruct((batch, seq_len, d_model), pos_table.dtype),
        scratch_shapes=[
            pltpu.VMEM((seq_len, d_model), pos_table.dtype),
            pltpu.SemaphoreType.DMA((_N_CHUNKS,)),
            pltpu.SemaphoreType.DMA((_N_CHUNKS, batch)),
        ],
    )(pos)


# speedup vs baseline: 68.7438x; 1.0131x over previous
import jax
import jax.numpy as jnp
from jax.experimental import pallas as pl
from jax.experimental.pallas import tpu as pltpu


_N_CHUNKS = 2


def _copy_body(pos_hbm, out_hbm, buf, in_sems, out_sems):
    batch = out_hbm.shape[0]
    seq_len = pos_hbm.shape[0]
    chunk = seq_len // _N_CHUNKS

    def in_copy(c):
        rows = pl.ds(c * chunk, chunk)
        return pltpu.make_async_copy(pos_hbm.at[rows], buf.at[rows], in_sems.at[c])

    def out_copy(c, b):
        rows = pl.ds(c * chunk, chunk)
        return pltpu.make_async_copy(buf.at[rows], out_hbm.at[b, rows], out_sems.at[c, b])

    for c in range(_N_CHUNKS):
        in_copy(c).start()
    for c in range(_N_CHUNKS):
        in_copy(c).wait()
        for b in range(batch):
            out_copy(c, b).start()
    for c in range(_N_CHUNKS):
        for b in range(batch):
            out_copy(c, b).wait()


def kernel(x, pos_table):
    batch, seq_len = x.shape
    d_model = pos_table.shape[1]
    pos = pos_table[:seq_len]
    return pl.pallas_call(
        _copy_body,
        in_specs=[pl.BlockSpec(memory_space=pl.ANY)],
        out_specs=pl.BlockSpec(memory_space=pl.ANY),
        out_shape=jax.ShapeDtypeStruct((batch, seq_len, d_model), pos_table.dtype),
        scratch_shapes=[
            pltpu.VMEM((seq_len, d_model), pos_table.dtype),
            pltpu.SemaphoreType.DMA((_N_CHUNKS,)),
            pltpu.SemaphoreType.DMA((_N_CHUNKS, batch)),
        ],
    )(pos)
